# baseline (speedup 1.0000x reference)
import functools

import jax
import jax.numpy as jnp
from jax import lax
from jax.experimental import pallas as pl
from jax.experimental.pallas import tpu as pltpu
from jax.experimental.pallas import tpu_sc as plsc

_CHUNK = 512
_NBUF = 2


@functools.cache
def _make_gather(n_total: int, dim: int):
    info = plsc.get_sparse_core_info()
    nc, ns = info.num_cores, info.num_subcores
    nw = nc * ns
    assert n_total % nw == 0
    n_per_w = n_total // nw
    assert n_per_w % (_CHUNK * _NBUF) == 0
    nchunk = n_per_w // _CHUNK

    mesh = plsc.VectorSubcoreMesh(core_axis_name="c", subcore_axis_name="s")

    @functools.partial(
        pl.kernel,
        mesh=mesh,
        out_type=jax.ShapeDtypeStruct((n_total, dim), jnp.float32),
        scratch_types=[
            pltpu.VMEM((n_per_w,), jnp.int32),
            pltpu.VMEM((_NBUF, _CHUNK, dim), jnp.float32),
            pltpu.SemaphoreType.DMA((_NBUF,)),
            pltpu.SemaphoreType.DMA((_NBUF,)),
        ],
        compiler_params=pltpu.CompilerParams(use_tc_tiling_on_sc=False),
    )
    def gather_kernel(idx_hbm, table_hbm, out_hbm, idx_v, rows_v, gsem, osem):
        wid = lax.axis_index("s") * nc + lax.axis_index("c")
        base = wid * n_per_w
        pltpu.sync_copy(idx_hbm.at[pl.ds(base, n_per_w)], idx_v)

        def start_gather(g, b):
            pltpu.async_copy(
                table_hbm.at[idx_v.at[pl.ds(g * _CHUNK, _CHUNK)]],
                rows_v.at[b],
                gsem.at[b],
            )

        def wait_gather(b):
            pltpu.make_async_copy(
                table_hbm.at[idx_v.at[pl.ds(0, _CHUNK)]],
                rows_v.at[b],
                gsem.at[b],
            ).wait()

        def start_out(g, b):
            pltpu.async_copy(
                rows_v.at[b],
                out_hbm.at[pl.ds(base + g * _CHUNK, _CHUNK)],
                osem.at[b],
            )

        def wait_out(b):
            pltpu.make_async_copy(
                rows_v.at[b],
                out_hbm.at[pl.ds(base, _CHUNK)],
                osem.at[b],
            ).wait()

        for b in range(_NBUF):
            start_gather(b, b)

        @pl.loop(0, nchunk, step=_NBUF)
        def _(outer):
            for b in range(_NBUF):
                g = outer + b
                wait_gather(b)
                start_out(g, b)
                nxt = g + _NBUF

                @pl.when(nxt < nchunk)
                def _():
                    wait_out(b)
                    start_gather(nxt, b)

        for b in range(_NBUF):
            wait_out(b)

    return gather_kernel


def kernel(ids, table):
    batch, n_fields = ids.shape
    _, dim = table.shape
    n_total = batch * n_fields
    ids_flat = ids.reshape(n_total).astype(jnp.int32)
    out = _make_gather(n_total, dim)(ids_flat, table)
    return out.reshape(batch, n_fields, dim)
